# baseline (speedup 1.0000x reference)
import functools

import jax
import jax.numpy as jnp
from jax import lax
from jax.experimental import pallas as pl
from jax.experimental.pallas import tpu as pltpu
from jax.experimental.pallas import tpu_sc as plsc

NC = 2
NS = 16
NW = NC * NS
LANES = 16
NPAD = 10240
ROWS_PT = NPAD // NW
GRP = 64
SB = 2000
PCAP = SB + GRP + LANES


def _spmm_sc(x, epack, n_pad, d):
    nsb = epack.shape[0]

    mesh = plsc.VectorSubcoreMesh(core_axis_name="c", subcore_axis_name="s")

    @functools.partial(
        pl.kernel,
        mesh=mesh,
        compiler_params=pltpu.CompilerParams(needs_layout_passes=False),
        out_type=jax.ShapeDtypeStruct((n_pad, d), jnp.float32),
        scratch_types=[
            pltpu.VMEM((3, SB), jnp.int32),
            pltpu.VMEM((PCAP,), jnp.int32),
            pltpu.VMEM((PCAP,), jnp.float32),
            pltpu.VMEM((PCAP,), jnp.int32),
            pltpu.VMEM((GRP, d), jnp.float32),
            pltpu.VMEM((GRP, d), jnp.float32),
            pltpu.VMEM((ROWS_PT, d), jnp.float32),
            pltpu.SemaphoreType.DMA,
            pltpu.SemaphoreType.DMA,
        ],
    )
    def spmm_kernel(x_hbm, ep_hbm, out_hbm,
                    eb3, p_src, p_w, p_loc, rows, rows_b, acc, sem, sem_b):
        cid = lax.axis_index("c")
        sid = lax.axis_index("s")
        tid = sid * NC + cid
        lo = tid * ROWS_PT

        zero = jnp.zeros((LANES,), jnp.float32)

        @plsc.parallel_loop(0, ROWS_PT)
        def zbody(r):
            for j in range(d // LANES):
                acc[r, pl.ds(j * LANES, LANES)] = zero

        def start_gather(g, buf, sm):
            pltpu.async_copy(x_hbm.at[p_src.at[pl.ds(g * GRP, GRP)]],
                             buf, sm)

        def wait_gather(buf, sm):
            pltpu.make_async_copy(x_hbm.at[p_src.at[pl.ds(0, GRP)]],
                                  buf, sm).wait()

        def accum_group(g, buf):
            nj = d // LANES

            @plsc.parallel_loop(0, GRP // LANES)
            def grp_body(q):
                o = g * GRP + q * LANES
                loc16 = p_loc[pl.ds(o, LANES)]
                locs = [loc16[r] for r in range(LANES)]
                nxt = [buf[q * LANES, pl.ds(j * LANES, LANES)]
                       for j in range(nj)]
                for r in range(LANES):
                    cur = nxt
                    wv = plsc.load_gather(
                        p_w, [jnp.full((LANES,), o + r, dtype=jnp.int32)])
                    if r + 1 < LANES:
                        nxt = [buf[q * LANES + r + 1,
                                   pl.ds(j * LANES, LANES)]
                               for j in range(nj)]
                    for j in range(nj):
                        sl = pl.ds(j * LANES, LANES)
                        plsc.addupdate(acc.at[locs[r], sl], wv * cur[j])

        def drain_group(g):
            start_gather(g, rows, sem)
            wait_gather(rows, sem)
            accum_group(g, rows)

        def sb_body(sb, base):
            pltpu.sync_copy(ep_hbm.at[sb], eb3)

            @plsc.parallel_loop(0, SB // LANES, carry=base)
            def scan_body(v, b):
                o = v * LANES
                s16 = eb3[0, pl.ds(o, LANES)]
                d16 = eb3[1, pl.ds(o, LANES)]
                w16 = plsc.bitcast(eb3[2, pl.ds(o, LANES)], jnp.float32)
                loc = d16 - lo
                m = (loc >= 0) & (loc < ROWS_PT)
                plsc.store_compressed(p_src.at[pl.ds(b, LANES)], s16, mask=m)
                plsc.store_compressed(p_w.at[pl.ds(b, LANES)], w16, mask=m)
                plsc.store_compressed(p_loc.at[pl.ds(b, LANES)], loc, mask=m)
                cnt = plsc.all_reduce_population_count(m)
                return b + cnt[0]
            base2 = scan_body

            ng = base2 // GRP

            @pl.when(ng > 0)
            def _():
                start_gather(0, rows, sem)

            def pair_body(t, _):
                g0 = 2 * t
                g1 = g0 + 1

                @pl.when(g1 < ng)
                def _():
                    start_gather(g1, rows_b, sem_b)
                wait_gather(rows, sem)
                accum_group(g0, rows)

                @pl.when(g1 < ng)
                def _():
                    @pl.when(g1 + 1 < ng)
                    def _():
                        start_gather(g1 + 1, rows, sem)
                    wait_gather(rows_b, sem_b)
                    accum_group(g1, rows_b)
                return 0
            lax.fori_loop(0, (ng + 1) // 2, pair_body, 0)

            for v in range(GRP // LANES):
                t = pl.ds(ng * GRP + v * LANES, LANES)
                f = pl.ds(v * LANES, LANES)
                s_t = p_src[t]
                w_t = p_w[t]
                l_t = p_loc[t]
                p_src[f] = s_t
                p_w[f] = w_t
                p_loc[f] = l_t
            return base2 - ng * GRP

        base_f = lax.fori_loop(0, nsb, sb_body, jnp.int32(0))

        iota16 = lax.iota(jnp.int32, LANES)
        for v in range(GRP // LANES):
            sl = pl.ds(v * LANES, LANES)
            keep = (iota16 + v * LANES) < base_f
            p_w[sl] = jnp.where(keep, p_w[sl], 0.0)
            p_src[sl] = jnp.where(keep, p_src[sl], 0)
            p_loc[sl] = jnp.where(keep, p_loc[sl], 0)
        drain_group(0)

        pltpu.sync_copy(acc, out_hbm.at[pl.ds(lo, ROWS_PT)])

    return spmm_kernel(x, epack)


def _mm_relu_tc(a, w0, bm=1024):
    m, d = a.shape

    def body(a_ref, w_ref, o_ref):
        o_ref[...] = jnp.maximum(
            lax.dot_general(a_ref[...], w_ref[...],
                            (((1,), (1,)), ((), ())),
                            preferred_element_type=jnp.float32),
            0.0)

    return pl.pallas_call(
        body,
        grid=(m // bm,),
        in_specs=[pl.BlockSpec((bm, d), lambda i: (i, 0)),
                  pl.BlockSpec((d, d), lambda i: (0, 0))],
        out_specs=pl.BlockSpec((bm, d), lambda i: (i, 0)),
        out_shape=jax.ShapeDtypeStruct((m, d), jnp.float32),
    )(a, w0)


def _mm_two_tc(c, w1, wss, bm=1024):
    m, d = c.shape
    ss = wss.shape[0]

    def body(c_ref, w1_ref, wss_ref, h_ref, hss_ref):
        cc = c_ref[...]
        h_ref[...] = lax.dot_general(cc, w1_ref[...],
                                     (((1,), (1,)), ((), ())),
                                     preferred_element_type=jnp.float32)
        hss_ref[...] = lax.dot_general(cc, wss_ref[...],
                                       (((1,), (1,)), ((), ())),
                                       preferred_element_type=jnp.float32)

    return pl.pallas_call(
        body,
        grid=(m // bm,),
        in_specs=[pl.BlockSpec((bm, d), lambda i: (i, 0)),
                  pl.BlockSpec((d, d), lambda i: (0, 0)),
                  pl.BlockSpec((ss, d), lambda i: (0, 0))],
        out_specs=[pl.BlockSpec((bm, d), lambda i: (i, 0)),
                   pl.BlockSpec((bm, ss), lambda i: (i, 0))],
        out_shape=[jax.ShapeDtypeStruct((m, d), jnp.float32),
                   jax.ShapeDtypeStruct((m, ss), jnp.float32)],
    )(c, w1, wss)


def kernel(x, edge_index, edge_weight, W0, W1, W_ss):
    n, d = x.shape
    w_bits = jax.lax.bitcast_convert_type(edge_weight, jnp.int32)
    epack = (jnp.stack([edge_index[0], edge_index[1], w_bits])
             .reshape(3, -1, SB).transpose(1, 0, 2))

    a = _spmm_sc(x, epack, NPAD, d)
    b = _mm_relu_tc(a, W0)
    c = _spmm_sc(b, epack, NPAD, d)
    h, h_ss = _mm_two_tc(c, W1, W_ss)
    return h[:n], h_ss[:n]


# speedup vs baseline: 1.1044x; 1.1044x over previous
import functools

import jax
import jax.numpy as jnp
from jax import lax
from jax.experimental import pallas as pl
from jax.experimental.pallas import tpu as pltpu
from jax.experimental.pallas import tpu_sc as plsc

NC = 2
NS = 16
NW = NC * NS
LANES = 16
NPAD = 10240
ROWS_PT = NPAD // NW
GRP = 80
SB = 2000
PCAP = SB + GRP + LANES


def _spmm_sc(x, epack, n_pad, d):
    nsb = epack.shape[0]

    mesh = plsc.VectorSubcoreMesh(core_axis_name="c", subcore_axis_name="s")

    @functools.partial(
        pl.kernel,
        mesh=mesh,
        compiler_params=pltpu.CompilerParams(needs_layout_passes=False),
        out_type=jax.ShapeDtypeStruct((n_pad, d), jnp.float32),
        scratch_types=[
            pltpu.VMEM((3, SB), jnp.int32),
            pltpu.VMEM((PCAP,), jnp.int32),
            pltpu.VMEM((PCAP,), jnp.float32),
            pltpu.VMEM((PCAP,), jnp.int32),
            pltpu.VMEM((GRP, d), jnp.float32),
            pltpu.VMEM((ROWS_PT, d), jnp.float32),
            pltpu.SemaphoreType.DMA,
        ],
    )
    def spmm_kernel(x_hbm, ep_hbm, out_hbm,
                    eb3, p_src, p_w, p_loc, rows, acc, sem):
        cid = lax.axis_index("c")
        sid = lax.axis_index("s")
        tid = sid * NC + cid
        lo = tid * ROWS_PT

        zero = jnp.zeros((LANES,), jnp.float32)

        @plsc.parallel_loop(0, ROWS_PT)
        def zbody(r):
            for j in range(d // LANES):
                acc[r, pl.ds(j * LANES, LANES)] = zero

        def drain_group(g):
            pltpu.async_copy(x_hbm.at[p_src.at[pl.ds(g * GRP, GRP)]],
                             rows, sem).wait()

            nj = d // LANES

            @plsc.parallel_loop(0, GRP // LANES)
            def grp_body(q):
                o = g * GRP + q * LANES
                loc16 = p_loc[pl.ds(o, LANES)]
                locs = [loc16[r] for r in range(LANES)]
                nxt = [rows[q * LANES, pl.ds(j * LANES, LANES)]
                       for j in range(nj)]
                for r in range(LANES):
                    cur = nxt
                    wv = plsc.load_gather(
                        p_w, [jnp.full((LANES,), o + r, dtype=jnp.int32)])
                    if r + 1 < LANES:
                        nxt = [rows[q * LANES + r + 1,
                                    pl.ds(j * LANES, LANES)]
                               for j in range(nj)]
                    for j in range(nj):
                        sl = pl.ds(j * LANES, LANES)
                        plsc.addupdate(acc.at[locs[r], sl], wv * cur[j])

        def sb_body(sb, base):
            pltpu.sync_copy(ep_hbm.at[sb], eb3)

            @plsc.parallel_loop(0, SB // LANES, carry=base)
            def scan_body(v, b):
                o = v * LANES
                s16 = eb3[0, pl.ds(o, LANES)]
                d16 = eb3[1, pl.ds(o, LANES)]
                w16 = plsc.bitcast(eb3[2, pl.ds(o, LANES)], jnp.float32)
                loc = d16 - lo
                m = (loc >= 0) & (loc < ROWS_PT)
                plsc.store_compressed(p_src.at[pl.ds(b, LANES)], s16, mask=m)
                plsc.store_compressed(p_w.at[pl.ds(b, LANES)], w16, mask=m)
                plsc.store_compressed(p_loc.at[pl.ds(b, LANES)], loc, mask=m)
                cnt = plsc.all_reduce_population_count(m)
                return b + cnt[0]
            base2 = scan_body

            ng = base2 // GRP

            def dg(g, _):
                drain_group(g)
                return 0
            lax.fori_loop(0, ng, dg, 0)

            for v in range(GRP // LANES):
                t = pl.ds(ng * GRP + v * LANES, LANES)
                f = pl.ds(v * LANES, LANES)
                s_t = p_src[t]
                w_t = p_w[t]
                l_t = p_loc[t]
                p_src[f] = s_t
                p_w[f] = w_t
                p_loc[f] = l_t
            return base2 - ng * GRP

        base_f = lax.fori_loop(0, nsb, sb_body, jnp.int32(0))

        iota16 = lax.iota(jnp.int32, LANES)
        for v in range(GRP // LANES):
            sl = pl.ds(v * LANES, LANES)
            keep = (iota16 + v * LANES) < base_f
            p_w[sl] = jnp.where(keep, p_w[sl], 0.0)
            p_src[sl] = jnp.where(keep, p_src[sl], 0)
            p_loc[sl] = jnp.where(keep, p_loc[sl], 0)
        drain_group(0)

        pltpu.sync_copy(acc, out_hbm.at[pl.ds(lo, ROWS_PT)])

    return spmm_kernel(x, epack)


def _mm_relu_tc(a, w0, bm=1024):
    m, d = a.shape

    def body(a_ref, w_ref, o_ref):
        o_ref[...] = jnp.maximum(
            lax.dot_general(a_ref[...], w_ref[...],
                            (((1,), (1,)), ((), ())),
                            preferred_element_type=jnp.float32),
            0.0)

    return pl.pallas_call(
        body,
        grid=(m // bm,),
        in_specs=[pl.BlockSpec((bm, d), lambda i: (i, 0)),
                  pl.BlockSpec((d, d), lambda i: (0, 0))],
        out_specs=pl.BlockSpec((bm, d), lambda i: (i, 0)),
        out_shape=jax.ShapeDtypeStruct((m, d), jnp.float32),
    )(a, w0)


def _mm_two_tc(c, w1, wss, bm=1024):
    m, d = c.shape
    ss = wss.shape[0]

    def body(c_ref, w1_ref, wss_ref, h_ref, hss_ref):
        cc = c_ref[...]
        h_ref[...] = lax.dot_general(cc, w1_ref[...],
                                     (((1,), (1,)), ((), ())),
                                     preferred_element_type=jnp.float32)
        hss_ref[...] = lax.dot_general(cc, wss_ref[...],
                                       (((1,), (1,)), ((), ())),
                                       preferred_element_type=jnp.float32)

    return pl.pallas_call(
        body,
        grid=(m // bm,),
        in_specs=[pl.BlockSpec((bm, d), lambda i: (i, 0)),
                  pl.BlockSpec((d, d), lambda i: (0, 0)),
                  pl.BlockSpec((ss, d), lambda i: (0, 0))],
        out_specs=[pl.BlockSpec((bm, d), lambda i: (i, 0)),
                   pl.BlockSpec((bm, ss), lambda i: (i, 0))],
        out_shape=[jax.ShapeDtypeStruct((m, d), jnp.float32),
                   jax.ShapeDtypeStruct((m, ss), jnp.float32)],
    )(c, w1, wss)


def kernel(x, edge_index, edge_weight, W0, W1, W_ss):
    n, d = x.shape
    w_bits = jax.lax.bitcast_convert_type(edge_weight, jnp.int32)
    epack = (jnp.stack([edge_index[0], edge_index[1], w_bits])
             .reshape(3, -1, SB).transpose(1, 0, 2))

    a = _spmm_sc(x, epack, NPAD, d)
    b = _mm_relu_tc(a, W0)
    c = _spmm_sc(b, epack, NPAD, d)
    h, h_ss = _mm_two_tc(c, W1, W_ss)
    return h[:n], h_ss[:n]
